# Initial kernel scaffold; baseline (speedup 1.0000x reference)
import jax
import jax.numpy as jnp
from jax.experimental import pallas as pl


def kernel(x, gate_W, gate_b, W1, b1, W2, b2, W3, b3):
    raise NotImplementedError("write your pallas kernel here")


# speedup vs baseline: 1.3251x; 1.3251x over previous
import functools

import jax
import jax.numpy as jnp
import numpy as np
from jax.experimental import pallas as pl
from jax.experimental.pallas import tpu as pltpu

_N_EXPERTS = 8
_N_ACTIVE = 5
_TEMPERATURE = float(np.e)
_BT = 512


def _moe_body(x_ref, gw_ref, gb_ref, w1_ref, b1_ref, w2_ref, b2_ref,
              w3_ref, b3_ref, out_ref, wts_ref):
    e = pl.program_id(0)
    tb = pl.program_id(1)
    tok = pl.ds(tb * _BT, _BT)
    xb = x_ref[...]

    @pl.when(e == 0)
    def _gate():
        s = jnp.dot(xb, gw_ref[...], preferred_element_type=jnp.float32)
        s = (s + gb_ref[...]) / _TEMPERATURE
        m = jnp.max(s, axis=-1, keepdims=True)
        p = jnp.exp(s - m)
        p = p / jnp.sum(p, axis=-1, keepdims=True)
        lane = jax.lax.broadcasted_iota(jnp.int32, p.shape, 1)
        sel = jnp.zeros(p.shape, dtype=jnp.bool_)
        pw = p
        for _ in range(_N_ACTIVE):
            mx = jnp.max(pw, axis=-1, keepdims=True)
            ismax = pw == mx
            midx = jnp.min(jnp.where(ismax, lane, _N_EXPERTS), axis=-1,
                           keepdims=True)
            first = lane == midx
            sel = jnp.logical_or(sel, first)
            pw = jnp.where(first, -1.0, pw)
        w = jnp.where(sel, p, 0.0)
        w = w / (jnp.sum(w, axis=-1, keepdims=True) + 1e-8)
        wts_ref[tok, :] = w

    wfull = wts_ref[tok, :]
    lane = jax.lax.broadcasted_iota(jnp.int32, wfull.shape, 1)
    w_col = jnp.sum(jnp.where(lane == e, wfull, 0.0), axis=-1,
                    keepdims=True)

    h1 = jnp.dot(xb, w1_ref[0], preferred_element_type=jnp.float32)
    h1 = jnp.maximum(h1 + b1_ref[0], 0.0)
    h2 = jnp.dot(h1, w2_ref[0], preferred_element_type=jnp.float32)
    h2 = jnp.maximum(h2 + b2_ref[0], 0.0)
    o = jnp.dot(h2, w3_ref[0], preferred_element_type=jnp.float32)
    o = o + b3_ref[0]
    contrib = o * w_col

    @pl.when(e == 0)
    def _init():
        out_ref[tok, :] = contrib

    @pl.when(e > 0)
    def _acc():
        out_ref[tok, :] += contrib


@jax.jit
def kernel(x, gate_W, gate_b, W1, b1, W2, b2, W3, b3):
    n, d = x.shape
    e, _, h = W1.shape
    o_dim = W3.shape[-1]
    grid = (e, n // _BT)
    out = pl.pallas_call(
        _moe_body,
        grid=grid,
        in_specs=[
            pl.BlockSpec((_BT, d), lambda ei, tb: (tb, 0)),
            pl.BlockSpec((d, _N_EXPERTS), lambda ei, tb: (0, 0)),
            pl.BlockSpec((1, _N_EXPERTS), lambda ei, tb: (0, 0)),
            pl.BlockSpec((1, d, h), lambda ei, tb: (ei, 0, 0)),
            pl.BlockSpec((1, 1, h), lambda ei, tb: (ei, 0, 0)),
            pl.BlockSpec((1, h, h), lambda ei, tb: (ei, 0, 0)),
            pl.BlockSpec((1, 1, h), lambda ei, tb: (ei, 0, 0)),
            pl.BlockSpec((1, h, o_dim), lambda ei, tb: (ei, 0, 0)),
            pl.BlockSpec((1, 1, o_dim), lambda ei, tb: (ei, 0, 0)),
        ],
        out_specs=pl.BlockSpec((n, o_dim), lambda ei, tb: (0, 0)),
        out_shape=jax.ShapeDtypeStruct((n, o_dim), jnp.float32),
        scratch_shapes=[pltpu.VMEM((n, _N_EXPERTS), jnp.float32)],
        compiler_params=pltpu.CompilerParams(
            dimension_semantics=("arbitrary", "arbitrary")),
    )(x, gate_W, gate_b.reshape(1, -1),
      W1, b1.reshape(e, 1, h), W2, b2.reshape(e, 1, h),
      W3, b3.reshape(e, 1, o_dim))
    return out


# speedup vs baseline: 1.3318x; 1.0051x over previous
import functools

import jax
import jax.numpy as jnp
import numpy as np
from jax.experimental import pallas as pl
from jax.experimental.pallas import tpu as pltpu

_N_EXPERTS = 8
_N_ACTIVE = 5
_TEMPERATURE = float(np.e)
_BT = 512


def _moe_body(x_ref, gw_ref, gb_ref, w1_ref, b1_ref, w2_ref, b2_ref,
              w3_ref, b3_ref, out_ref, wts_ref):
    e = pl.program_id(0)
    tb = pl.program_id(1)
    tok = pl.ds(tb * _BT, _BT)
    xb = x_ref[...]

    @pl.when(e == 0)
    def _gate():
        s = jnp.dot(xb, gw_ref[...], preferred_element_type=jnp.float32)
        s = (s + gb_ref[...]) / _TEMPERATURE
        m = jnp.max(s, axis=-1, keepdims=True)
        p = jnp.exp(s - m)
        p = p / jnp.sum(p, axis=-1, keepdims=True)
        lane = jax.lax.broadcasted_iota(jnp.int32, p.shape, 1)
        sel = jnp.zeros(p.shape, dtype=jnp.bool_)
        pw = p
        for _ in range(_N_ACTIVE):
            mx = jnp.max(pw, axis=-1, keepdims=True)
            ismax = pw == mx
            midx = jnp.min(jnp.where(ismax, lane, _N_EXPERTS), axis=-1,
                           keepdims=True)
            first = lane == midx
            sel = jnp.logical_or(sel, first)
            pw = jnp.where(first, -1.0, pw)
        w = jnp.where(sel, p, 0.0)
        w = w / (jnp.sum(w, axis=-1, keepdims=True) + 1e-8)
        wts_ref[tok, :] = w

    wfull = wts_ref[tok, :]
    lane = jax.lax.broadcasted_iota(jnp.int32, wfull.shape, 1)
    w_col = jnp.sum(jnp.where(lane == e, wfull, 0.0), axis=-1,
                    keepdims=True)

    bf = jnp.bfloat16
    h1 = jnp.dot(xb.astype(bf), w1_ref[0].astype(bf),
                 preferred_element_type=jnp.float32)
    h1 = jnp.maximum(h1 + b1_ref[0], 0.0)
    h2 = jnp.dot(h1.astype(bf), w2_ref[0].astype(bf),
                 preferred_element_type=jnp.float32)
    h2 = jnp.maximum(h2 + b2_ref[0], 0.0)
    o = jnp.dot(h2.astype(bf), w3_ref[0].astype(bf),
                preferred_element_type=jnp.float32)
    o = o + b3_ref[0]
    contrib = o * w_col

    @pl.when(e == 0)
    def _init():
        out_ref[tok, :] = contrib

    @pl.when(e > 0)
    def _acc():
        out_ref[tok, :] += contrib


@jax.jit
def kernel(x, gate_W, gate_b, W1, b1, W2, b2, W3, b3):
    n, d = x.shape
    e, _, h = W1.shape
    o_dim = W3.shape[-1]
    grid = (e, n // _BT)
    out = pl.pallas_call(
        _moe_body,
        grid=grid,
        in_specs=[
            pl.BlockSpec((_BT, d), lambda ei, tb: (tb, 0)),
            pl.BlockSpec((d, _N_EXPERTS), lambda ei, tb: (0, 0)),
            pl.BlockSpec((1, _N_EXPERTS), lambda ei, tb: (0, 0)),
            pl.BlockSpec((1, d, h), lambda ei, tb: (ei, 0, 0)),
            pl.BlockSpec((1, 1, h), lambda ei, tb: (ei, 0, 0)),
            pl.BlockSpec((1, h, h), lambda ei, tb: (ei, 0, 0)),
            pl.BlockSpec((1, 1, h), lambda ei, tb: (ei, 0, 0)),
            pl.BlockSpec((1, h, o_dim), lambda ei, tb: (ei, 0, 0)),
            pl.BlockSpec((1, 1, o_dim), lambda ei, tb: (ei, 0, 0)),
        ],
        out_specs=pl.BlockSpec((n, o_dim), lambda ei, tb: (0, 0)),
        out_shape=jax.ShapeDtypeStruct((n, o_dim), jnp.float32),
        scratch_shapes=[pltpu.VMEM((n, _N_EXPERTS), jnp.float32)],
        compiler_params=pltpu.CompilerParams(
            dimension_semantics=("arbitrary", "arbitrary")),
    )(x, gate_W, gate_b.reshape(1, -1),
      W1, b1.reshape(e, 1, h), W2, b2.reshape(e, 1, h),
      W3, b3.reshape(e, 1, o_dim))
    return out


# speedup vs baseline: 1.3616x; 1.0224x over previous
import functools

import jax
import jax.numpy as jnp
import numpy as np
from jax.experimental import pallas as pl
from jax.experimental.pallas import tpu as pltpu

_N_EXPERTS = 8
_N_ACTIVE = 5
_TEMPERATURE = float(np.e)
_BT = 512


def _moe_body(x_ref, gw_ref, gb_ref, w1_ref, b1_ref, w2_ref, b2_ref,
              w3_ref, b3_ref, out_ref, wts_ref, xbf_ref,
              w1b_ref, w2b_ref, w3b_ref):
    e = pl.program_id(0)
    tb = pl.program_id(1)
    tok = pl.ds(tb * _BT, _BT)
    bf = jnp.bfloat16

    @pl.when(e == 0)
    def _gate():
        xb = x_ref[...]
        xbf_ref[tok, :] = xb.astype(bf)
        s = jnp.dot(xb, gw_ref[...], preferred_element_type=jnp.float32)
        s = (s + gb_ref[...]) / _TEMPERATURE
        m = jnp.max(s, axis=-1, keepdims=True)
        p = jnp.exp(s - m)
        p = p / jnp.sum(p, axis=-1, keepdims=True)
        lane = jax.lax.broadcasted_iota(jnp.int32, p.shape, 1)
        sel = jnp.zeros(p.shape, dtype=jnp.bool_)
        pw = p
        for _ in range(_N_ACTIVE):
            mx = jnp.max(pw, axis=-1, keepdims=True)
            ismax = pw == mx
            midx = jnp.min(jnp.where(ismax, lane, _N_EXPERTS), axis=-1,
                           keepdims=True)
            first = lane == midx
            sel = jnp.logical_or(sel, first)
            pw = jnp.where(first, -1.0, pw)
        w = jnp.where(sel, p, 0.0)
        w = w / (jnp.sum(w, axis=-1, keepdims=True) + 1e-8)
        wts_ref[tok, :] = w

    @pl.when(tb == 0)
    def _cast_weights():
        w1b_ref[...] = w1_ref[0].astype(bf)
        w2b_ref[...] = w2_ref[0].astype(bf)
        w3b_ref[...] = w3_ref[0].astype(bf)

    wfull = wts_ref[tok, :]
    lane = jax.lax.broadcasted_iota(jnp.int32, wfull.shape, 1)
    w_col = jnp.sum(jnp.where(lane == e, wfull, 0.0), axis=-1,
                    keepdims=True)

    xb16 = xbf_ref[tok, :]
    h1 = jnp.dot(xb16, w1b_ref[...], preferred_element_type=jnp.float32)
    h1 = jnp.maximum(h1 + b1_ref[0], 0.0)
    h2 = jnp.dot(h1.astype(bf), w2b_ref[...],
                 preferred_element_type=jnp.float32)
    h2 = jnp.maximum(h2 + b2_ref[0], 0.0)
    o = jnp.dot(h2.astype(bf), w3b_ref[...],
                preferred_element_type=jnp.float32)
    o = o + b3_ref[0]
    contrib = o * w_col

    @pl.when(e == 0)
    def _init():
        out_ref[tok, :] = contrib

    @pl.when(e > 0)
    def _acc():
        out_ref[tok, :] += contrib


@jax.jit
def kernel(x, gate_W, gate_b, W1, b1, W2, b2, W3, b3):
    n, d = x.shape
    e, _, h = W1.shape
    o_dim = W3.shape[-1]
    ntb = n // _BT
    grid = (e, ntb)

    def x_map(ei, tb):
        return (jnp.where(ei == 0, tb, ntb - 1), 0)

    out = pl.pallas_call(
        _moe_body,
        grid=grid,
        in_specs=[
            pl.BlockSpec((_BT, d), x_map),
            pl.BlockSpec((d, _N_EXPERTS), lambda ei, tb: (0, 0)),
            pl.BlockSpec((1, _N_EXPERTS), lambda ei, tb: (0, 0)),
            pl.BlockSpec((1, d, h), lambda ei, tb: (ei, 0, 0)),
            pl.BlockSpec((1, 1, h), lambda ei, tb: (ei, 0, 0)),
            pl.BlockSpec((1, h, h), lambda ei, tb: (ei, 0, 0)),
            pl.BlockSpec((1, 1, h), lambda ei, tb: (ei, 0, 0)),
            pl.BlockSpec((1, h, o_dim), lambda ei, tb: (ei, 0, 0)),
            pl.BlockSpec((1, 1, o_dim), lambda ei, tb: (ei, 0, 0)),
        ],
        out_specs=pl.BlockSpec((n, o_dim), lambda ei, tb: (0, 0)),
        out_shape=jax.ShapeDtypeStruct((n, o_dim), jnp.float32),
        scratch_shapes=[
            pltpu.VMEM((n, _N_EXPERTS), jnp.float32),
            pltpu.VMEM((n, d), jnp.bfloat16),
            pltpu.VMEM((d, h), jnp.bfloat16),
            pltpu.VMEM((h, h), jnp.bfloat16),
            pltpu.VMEM((h, o_dim), jnp.bfloat16),
        ],
        compiler_params=pltpu.CompilerParams(
            dimension_semantics=("arbitrary", "arbitrary")),
    )(x, gate_W, gate_b.reshape(1, -1),
      W1, b1.reshape(e, 1, h), W2, b2.reshape(e, 1, h),
      W3, b3.reshape(e, 1, o_dim))
    return out
